# baseline (speedup 1.0000x reference)
import functools
import jax
import jax.numpy as jnp
from jax import lax
from jax.experimental import pallas as pl
from jax.experimental.pallas import tpu as pltpu
from jax.experimental.pallas import tpu_sc as plsc

_NUM_CLASSES = 5
_FEAT = 64
_LAMBDA_C = 0.01
_L = 16
_NW = 32
_CHUNKS = _FEAT // _L


def _lane_shuffle(x, idx):
    dnums = lax.GatherDimensionNumbers(
        offset_dims=(), collapsed_slice_dims=(0,), start_index_map=(0,))
    return lax.gather(x, idx[:, None], dnums, (1,),
                      mode=lax.GatherScatterMode.PROMISE_IN_BOUNDS)


def _allreduce_lanes(x):
    iota = lax.iota(jnp.int32, _L)
    for sh in (8, 4, 2, 1):
        x = x + _lane_shuffle(x, (iota + sh) % _L)
    return x


def _sc_body(z_hbm, t_hbm, c_hbm, out_hbm, z_v, t_v, c_v, csq_v, s_v, part_v,
             red_v, out_v, shared, sem):
    batch = t_hbm.shape[0]
    rows = batch // _NW
    groups = rows // _L
    wid = lax.axis_index("s") * 2 + lax.axis_index("c")

    zcopy = pltpu.make_async_copy(
        z_hbm.at[pl.ds(wid * rows, rows), :], z_v, sem)
    zcopy.start()
    pltpu.sync_copy(t_hbm.at[pl.ds(wid * rows, rows)], t_v)
    pltpu.sync_copy(c_hbm, c_v)

    iota = lax.iota(jnp.int32, _L)
    fzero = jnp.zeros((_L,), jnp.float32)

    csq = fzero
    for k in range(_NUM_CLASSES):
        sq = fzero
        for c in range(_CHUNKS):
            cv = c_v[pl.ds(k * _FEAT + c * _L, _L)]
            sq = sq + cv * cv
        csq = jnp.where(iota == k, _allreduce_lanes(sq), csq)
    csq_v[...] = csq

    for c in range(_NUM_CLASSES * _CHUNKS):
        s_v[pl.ds(c * _L, _L)] = fzero

    zcopy.wait()

    @plsc.parallel_loop(0, rows, unroll=16, carry=(fzero,) * _CHUNKS)
    def row_loop(row, accs):
        accs = list(accs)
        t_splat = plsc.load_gather(t_v, [jnp.full((_L,), row, jnp.int32)])
        cbase = t_splat * _FEAT + iota
        for c in range(_CHUNKS):
            zv = z_v[row, pl.ds(c * _L, _L)]
            accs[c] = accs[c] + zv * zv
            plsc.addupdate_scatter(s_v, [cbase + (c * _L)], zv)
        return tuple(accs)

    acc_sq = fzero
    for a in row_loop:
        acc_sq = acc_sq + a

    @plsc.parallel_loop(0, groups, unroll=4, carry=fzero)
    def csq_loop(g, acc):
        t_vec = t_v[pl.ds(g * _L, _L)]
        return acc + plsc.load_gather(csq_v, [t_vec])

    acc_csq = csq_loop

    cross = fzero
    for k in range(_NUM_CLASSES):
        for c in range(_CHUNKS):
            o = k * _FEAT + c * _L
            cross = cross + s_v[pl.ds(o, _L)] * c_v[pl.ds(o, _L)]

    sid = lax.axis_index("s")
    cid = lax.axis_index("c")
    scale = _LAMBDA_C * 0.5 / batch
    part_v[...] = (acc_sq - 2.0 * cross + acc_csq) * scale
    pltpu.sync_copy(part_v, shared.at[pl.ds(sid * _L, _L)])
    plsc.subcore_barrier()

    @pl.when(sid == 0)
    def _finish():
        pltpu.sync_copy(shared, red_v)
        total = fzero
        for w in range(_NW // 2):
            total = total + red_v[pl.ds(w * _L, _L)]
        out_v[...] = total
        pltpu.sync_copy(out_v, out_hbm.at[cid])


def kernel(z, targets, centers):
    batch = z.shape[0]
    rows = batch // _NW
    run = pl.kernel(
        _sc_body,
        out_type=jax.ShapeDtypeStruct((2, _L), jnp.float32),
        mesh=plsc.VectorSubcoreMesh(core_axis_name="c", subcore_axis_name="s"),
        compiler_params=pltpu.CompilerParams(needs_layout_passes=False),
        scratch_types=[
            pltpu.VMEM((rows, _FEAT), jnp.float32),
            pltpu.VMEM((rows,), jnp.int32),
            pltpu.VMEM((_NUM_CLASSES * _FEAT,), jnp.float32),
            pltpu.VMEM((_L,), jnp.float32),
            pltpu.VMEM((_NUM_CLASSES * _FEAT,), jnp.float32),
            pltpu.VMEM((_L,), jnp.float32),
            pltpu.VMEM((_NW // 2 * _L,), jnp.float32),
            pltpu.VMEM((_L,), jnp.float32),
            pltpu.VMEM_SHARED((_NW // 2 * _L,), jnp.float32),
            pltpu.SemaphoreType.DMA,
        ],
    )
    out = run(z, targets.astype(jnp.int32),
              centers.reshape(_NUM_CLASSES * _FEAT))
    return jnp.sum(out)


# speedup vs baseline: 1.0058x; 1.0058x over previous
import functools
import jax
import jax.numpy as jnp
from jax import lax
from jax.experimental import pallas as pl
from jax.experimental.pallas import tpu as pltpu
from jax.experimental.pallas import tpu_sc as plsc

_NUM_CLASSES = 5
_FEAT = 64
_LAMBDA_C = 0.01
_L = 16
_NW = 32
_CHUNKS = _FEAT // _L


def _lane_shuffle(x, idx):
    dnums = lax.GatherDimensionNumbers(
        offset_dims=(), collapsed_slice_dims=(0,), start_index_map=(0,))
    return lax.gather(x, idx[:, None], dnums, (1,),
                      mode=lax.GatherScatterMode.PROMISE_IN_BOUNDS)


def _allreduce_lanes(x):
    iota = lax.iota(jnp.int32, _L)
    for sh in (8, 4, 2, 1):
        x = x + _lane_shuffle(x, (iota + sh) % _L)
    return x


def _sc_body(z_hbm, t_hbm, c_hbm, out_hbm, z_v, t_v, c_v, csq_v, s_v, part_v,
             red_v, out_v, shared, sem):
    batch = t_hbm.shape[0]
    rows = batch // _NW
    groups = rows // _L
    wid = lax.axis_index("s") * 2 + lax.axis_index("c")

    zcopy = pltpu.make_async_copy(
        z_hbm.at[pl.ds(wid * rows, rows), :], z_v, sem)
    zcopy.start()
    pltpu.sync_copy(t_hbm.at[pl.ds(wid * rows, rows)], t_v)
    pltpu.sync_copy(c_hbm, c_v)

    iota = lax.iota(jnp.int32, _L)
    fzero = jnp.zeros((_L,), jnp.float32)

    csq = fzero
    for k in range(_NUM_CLASSES):
        sq = fzero
        for c in range(_CHUNKS):
            cv = c_v[pl.ds(k * _FEAT + c * _L, _L)]
            sq = sq + cv * cv
        csq = jnp.where(iota == k, _allreduce_lanes(sq), csq)
    csq_v[...] = csq

    for c in range(_NUM_CLASSES * _CHUNKS):
        s_v[pl.ds(c * _L, _L)] = fzero

    zcopy.wait()

    @plsc.parallel_loop(0, rows, unroll=8, carry=(fzero,) * _CHUNKS)
    def row_loop(row, accs):
        accs = list(accs)
        t_splat = plsc.load_gather(t_v, [jnp.full((_L,), row, jnp.int32)])
        cbase = t_splat * _FEAT + iota
        for c in range(_CHUNKS):
            zv = z_v[row, pl.ds(c * _L, _L)]
            accs[c] = accs[c] + zv * zv
            plsc.addupdate_scatter(s_v, [cbase + (c * _L)], zv)
        return tuple(accs)

    acc_sq = fzero
    for a in row_loop:
        acc_sq = acc_sq + a

    @plsc.parallel_loop(0, groups, unroll=4, carry=fzero)
    def csq_loop(g, acc):
        t_vec = t_v[pl.ds(g * _L, _L)]
        return acc + plsc.load_gather(csq_v, [t_vec])

    acc_csq = csq_loop

    cross = fzero
    for k in range(_NUM_CLASSES):
        for c in range(_CHUNKS):
            o = k * _FEAT + c * _L
            cross = cross + s_v[pl.ds(o, _L)] * c_v[pl.ds(o, _L)]

    sid = lax.axis_index("s")
    cid = lax.axis_index("c")
    scale = _LAMBDA_C * 0.5 / batch
    part_v[...] = (acc_sq - 2.0 * cross + acc_csq) * scale
    pltpu.sync_copy(part_v, shared.at[pl.ds(sid * _L, _L)])
    plsc.subcore_barrier()

    @pl.when(sid == 0)
    def _finish():
        pltpu.sync_copy(shared, red_v)
        total = fzero
        for w in range(_NW // 2):
            total = total + red_v[pl.ds(w * _L, _L)]
        out_v[...] = total
        pltpu.sync_copy(out_v, out_hbm.at[cid])


def kernel(z, targets, centers):
    batch = z.shape[0]
    rows = batch // _NW
    run = pl.kernel(
        _sc_body,
        out_type=jax.ShapeDtypeStruct((2, _L), jnp.float32),
        mesh=plsc.VectorSubcoreMesh(core_axis_name="c", subcore_axis_name="s"),
        compiler_params=pltpu.CompilerParams(needs_layout_passes=False, use_tc_tiling_on_sc=True),
        scratch_types=[
            pltpu.VMEM((rows, _FEAT), jnp.float32),
            pltpu.VMEM((rows,), jnp.int32),
            pltpu.VMEM((_NUM_CLASSES * _FEAT,), jnp.float32),
            pltpu.VMEM((_L,), jnp.float32),
            pltpu.VMEM((_NUM_CLASSES * _FEAT,), jnp.float32),
            pltpu.VMEM((_L,), jnp.float32),
            pltpu.VMEM((_NW // 2 * _L,), jnp.float32),
            pltpu.VMEM((_L,), jnp.float32),
            pltpu.VMEM_SHARED((_NW // 2 * _L,), jnp.float32),
            pltpu.SemaphoreType.DMA,
        ],
    )
    out = run(z, targets.astype(jnp.int32),
              centers.reshape(_NUM_CLASSES * _FEAT))
    return jnp.sum(out)


# speedup vs baseline: 1.0466x; 1.0405x over previous
import functools
import jax
import jax.numpy as jnp
from jax import lax
from jax.experimental import pallas as pl
from jax.experimental.pallas import tpu as pltpu
from jax.experimental.pallas import tpu_sc as plsc

_NUM_CLASSES = 5
_FEAT = 64
_LAMBDA_C = 0.01
_L = 16
_NW = 32
_CHUNKS = _FEAT // _L


def _lane_shuffle(x, idx):
    dnums = lax.GatherDimensionNumbers(
        offset_dims=(), collapsed_slice_dims=(0,), start_index_map=(0,))
    return lax.gather(x, idx[:, None], dnums, (1,),
                      mode=lax.GatherScatterMode.PROMISE_IN_BOUNDS)


def _allreduce_lanes(x):
    iota = lax.iota(jnp.int32, _L)
    for sh in (8, 4, 2, 1):
        x = x + _lane_shuffle(x, (iota + sh) % _L)
    return x


def _sc_body(z_hbm, t_hbm, c_hbm, out_hbm, z_v, t_v, c_v, csq_v, s_v, part_v,
             red_v, out_v, shared, sem):
    batch = t_hbm.shape[0]
    rows = batch // _NW
    groups = rows // _L
    wid = lax.axis_index("s") * 2 + lax.axis_index("c")

    zcopy = pltpu.make_async_copy(
        z_hbm.at[pl.ds(wid * rows, rows), :], z_v, sem)
    zcopy.start()
    pltpu.sync_copy(t_hbm.at[pl.ds(wid * rows, rows)], t_v)
    pltpu.sync_copy(c_hbm, c_v)

    iota = lax.iota(jnp.int32, _L)
    fzero = jnp.zeros((_L,), jnp.float32)

    csq = fzero
    for k in range(_NUM_CLASSES):
        sq = fzero
        for c in range(_CHUNKS):
            cv = c_v[k, pl.ds(c * _L, _L)]
            sq = sq + cv * cv
        csq = jnp.where(iota == k, _allreduce_lanes(sq), csq)
    csq_v[...] = csq

    for c in range(_NUM_CLASSES * _CHUNKS):
        s_v[pl.ds(c * _L, _L)] = fzero

    zcopy.wait()

    @plsc.parallel_loop(0, rows, unroll=8, carry=(fzero,) * _CHUNKS)
    def row_loop(row, accs):
        accs = list(accs)
        t_splat = plsc.load_gather(t_v, [jnp.full((_L,), row, jnp.int32)])
        cbase = t_splat * _FEAT + iota
        for c in range(_CHUNKS):
            zv = z_v[row, pl.ds(c * _L, _L)]
            accs[c] = accs[c] + zv * zv
            plsc.addupdate_scatter(s_v, [cbase + (c * _L)], zv)
        return tuple(accs)

    acc_sq = fzero
    for a in row_loop:
        acc_sq = acc_sq + a

    @plsc.parallel_loop(0, groups, unroll=4, carry=fzero)
    def csq_loop(g, acc):
        t_vec = t_v[pl.ds(g * _L, _L)]
        return acc + plsc.load_gather(csq_v, [t_vec])

    acc_csq = csq_loop

    cross = fzero
    for k in range(_NUM_CLASSES):
        for c in range(_CHUNKS):
            o = k * _FEAT + c * _L
            cross = cross + s_v[pl.ds(o, _L)] * c_v[k, pl.ds(c * _L, _L)]

    sid = lax.axis_index("s")
    cid = lax.axis_index("c")
    scale = _LAMBDA_C * 0.5 / batch
    part_v[...] = (acc_sq - 2.0 * cross + acc_csq) * scale
    pltpu.sync_copy(part_v, shared.at[pl.ds(sid * _L, _L)])
    plsc.subcore_barrier()

    @pl.when(sid == 0)
    def _finish():
        pltpu.sync_copy(shared, red_v)
        total = fzero
        for w in range(_NW // 2):
            total = total + red_v[pl.ds(w * _L, _L)]
        out_v[...] = total
        pltpu.sync_copy(out_v, out_hbm.at[cid])


def kernel(z, targets, centers):
    batch = z.shape[0]
    rows = batch // _NW
    run = pl.kernel(
        _sc_body,
        out_type=jax.ShapeDtypeStruct((2, _L), jnp.float32),
        mesh=plsc.VectorSubcoreMesh(core_axis_name="c", subcore_axis_name="s"),
        compiler_params=pltpu.CompilerParams(needs_layout_passes=False),
        scratch_types=[
            pltpu.VMEM((rows, _FEAT), jnp.float32),
            pltpu.VMEM((rows,), jnp.int32),
            pltpu.VMEM((_NUM_CLASSES, _FEAT), jnp.float32),
            pltpu.VMEM((_L,), jnp.float32),
            pltpu.VMEM((_NUM_CLASSES * _FEAT,), jnp.float32),
            pltpu.VMEM((_L,), jnp.float32),
            pltpu.VMEM((_NW // 2 * _L,), jnp.float32),
            pltpu.VMEM((_L,), jnp.float32),
            pltpu.VMEM_SHARED((_NW // 2 * _L,), jnp.float32),
            pltpu.SemaphoreType.DMA,
        ],
    )
    out = run(z, targets.astype(jnp.int32), centers)
    return jnp.sum(out)
